# Initial kernel scaffold; baseline (speedup 1.0000x reference)
import jax
import jax.numpy as jnp
from jax.experimental import pallas as pl


def kernel(x, edge_index, W1, b1, W2, b2):
    raise NotImplementedError("write your pallas kernel here")


# speedup vs baseline: 12.2496x; 12.2496x over previous
import functools

import jax
import jax.numpy as jnp
from jax import lax
from jax.experimental import pallas as pl
from jax.experimental.pallas import tpu as pltpu
from jax.experimental.pallas import tpu_sc as plsc

N = 10000
E = 320000
D = 128

NC = 2
NS = 16
NW = NC * NS
EPW = E // NW
CH = 80
NCH = EPW // CH
RB = 1000

_mesh = plsc.VectorSubcoreMesh(core_axis_name="c", subcore_axis_name="s")


def _deg_body(dst_hbm, ones_hbm, zeros_hbm, out_hbm, ones_v, idx_v, acc_sh):
    c = lax.axis_index("c")
    s = lax.axis_index("s")
    w = c * NS + s
    @pl.when(s < N // RB)
    def _():
        pltpu.sync_copy(zeros_hbm, acc_sh.at[pl.ds(s * RB, RB)])
    pltpu.sync_copy(ones_hbm, ones_v)
    plsc.subcore_barrier()

    def chunk(j, carry):
        off = w * EPW + j * CH
        pltpu.sync_copy(dst_hbm.at[pl.ds(off, CH)], idx_v)
        pltpu.sync_copy(ones_v, acc_sh.at[idx_v], add=True)
        return carry

    lax.fori_loop(0, NCH, chunk, 0)
    plsc.subcore_barrier()

    @pl.when(s < N // RB)
    def _():
        pltpu.sync_copy(acc_sh.at[pl.ds(s * RB, RB)],
                        out_hbm.at[c, pl.ds(s * RB, RB)])


_deg_call = pl.kernel(
    _deg_body,
    out_type=jax.ShapeDtypeStruct((NC, N, D), jnp.float32),
    mesh=_mesh,
    scratch_types=[
        pltpu.VMEM((CH, D), jnp.float32),
        pltpu.VMEM((CH,), jnp.int32),
        pltpu.VMEM_SHARED((N, D), jnp.float32),
    ],
)


def _spmm_body(xws_hbm, src_hbm, dst_hbm, zeros_hbm, out_hbm,
               idx_s, idx_d, rows_v, acc_sh, gsem):
    c = lax.axis_index("c")
    s = lax.axis_index("s")
    w = c * NS + s

    @pl.when(s < N // RB)
    def _():
        pltpu.sync_copy(zeros_hbm, acc_sh.at[pl.ds(s * RB, RB)])

    plsc.subcore_barrier()

    def chunk(j, carry):
        off = w * EPW + j * CH
        pltpu.sync_copy(src_hbm.at[pl.ds(off, CH)], idx_s)
        pltpu.sync_copy(dst_hbm.at[pl.ds(off, CH)], idx_d)
        pltpu.async_copy(xws_hbm.at[idx_s], rows_v, gsem).wait()
        pltpu.sync_copy(rows_v, acc_sh.at[idx_d], add=True)
        return carry

    lax.fori_loop(0, NCH, chunk, 0)
    plsc.subcore_barrier()

    @pl.when(s < N // RB)
    def _():
        pltpu.sync_copy(acc_sh.at[pl.ds(s * RB, RB)],
                        out_hbm.at[c, pl.ds(s * RB, RB)])


_spmm_call = pl.kernel(
    _spmm_body,
    out_type=jax.ShapeDtypeStruct((NC, N, D), jnp.float32),
    mesh=_mesh,
    scratch_types=[
        pltpu.VMEM((CH,), jnp.int32),
        pltpu.VMEM((CH,), jnp.int32),
        pltpu.VMEM((CH, D), jnp.float32),
        pltpu.VMEM_SHARED((N, D), jnp.float32),
        pltpu.SemaphoreType.DMA,
    ],
)


_R = 1000


def _dis(d0_ref, d1_ref):
    deg = d0_ref[:, 0:1] + d1_ref[:, 0:1] + 1.0
    return lax.rsqrt(deg)


def _pre_body(x_ref, w_ref, d0_ref, d1_ref, o_ref):
    dis = _dis(d0_ref, d1_ref)
    xw = jnp.dot(x_ref[...], w_ref[...], preferred_element_type=jnp.float32)
    o_ref[...] = dis * xw


def _mid_body(a0_ref, a1_ref, xws_ref, d0_ref, d1_ref, b_ref, w_ref, o_ref):
    dis = _dis(d0_ref, d1_ref)
    pre = dis * (a0_ref[...] + a1_ref[...] + xws_ref[...]) + b_ref[...]
    h = jnp.maximum(pre, 0.0)
    o_ref[...] = dis * jnp.dot(h, w_ref[...],
                               preferred_element_type=jnp.float32)


def _post_body(a0_ref, a1_ref, xws_ref, d0_ref, d1_ref, b_ref, o_ref):
    dis = _dis(d0_ref, d1_ref)
    o_ref[...] = dis * (a0_ref[...] + a1_ref[...] + xws_ref[...]) + b_ref[...]


_row_spec = pl.BlockSpec((_R, D), lambda i: (i, 0))
_deg_spec = pl.BlockSpec((_R, D), lambda i: (i, 0))
_w_spec = pl.BlockSpec((D, D), lambda i: (0, 0))
_b_spec = pl.BlockSpec((1, D), lambda i: (0, 0))
_grid = (N // _R,)

_pre_call = pl.pallas_call(
    _pre_body,
    grid=_grid,
    in_specs=[_row_spec, _w_spec, _deg_spec, _deg_spec],
    out_specs=_row_spec,
    out_shape=jax.ShapeDtypeStruct((N, D), jnp.float32),
)

_mid_call = pl.pallas_call(
    _mid_body,
    grid=_grid,
    in_specs=[_row_spec, _row_spec, _row_spec, _deg_spec, _deg_spec,
              _b_spec, _w_spec],
    out_specs=_row_spec,
    out_shape=jax.ShapeDtypeStruct((N, D), jnp.float32),
)

_post_call = pl.pallas_call(
    _post_body,
    grid=_grid,
    in_specs=[_row_spec, _row_spec, _row_spec, _deg_spec, _deg_spec, _b_spec],
    out_specs=_row_spec,
    out_shape=jax.ShapeDtypeStruct((N, D), jnp.float32),
)


def kernel(x, edge_index, W1, b1, W2, b2):
    ei = edge_index.astype(jnp.int32)
    src, dst = ei[0], ei[1]
    onesD = jnp.ones((CH, D), jnp.float32)
    zerosD = jnp.zeros((RB, D), jnp.float32)
    b1r = b1.reshape(1, D)
    b2r = b2.reshape(1, D)

    degp = _deg_call(dst, onesD, zerosD)
    d0, d1 = degp[0], degp[1]

    xw1s = _pre_call(x, W1, d0, d1)
    acc1 = _spmm_call(xw1s, src, dst, zerosD)
    xw2s = _mid_call(acc1[0], acc1[1], xw1s, d0, d1, b1r, W2)
    acc2 = _spmm_call(xw2s, src, dst, zerosD)
    return _post_call(acc2[0], acc2[1], xw2s, d0, d1, b2r)


# speedup vs baseline: 19.4174x; 1.5851x over previous
import functools

import jax
import jax.numpy as jnp
from jax import lax
from jax.experimental import pallas as pl
from jax.experimental.pallas import tpu as pltpu
from jax.experimental.pallas import tpu_sc as plsc

N = 10000
E = 320000
D = 128

NC = 2
NS = 16
NW = NC * NS
EPW = E // NW
CH = 80
NCH = EPW // CH
RB = 1000

_mesh = plsc.VectorSubcoreMesh(core_axis_name="c", subcore_axis_name="s")


def _deg_body(dst_hbm, ones_hbm, zeros_hbm, out_hbm, ones_v, idx_v, acc_sh):
    c = lax.axis_index("c")
    s = lax.axis_index("s")
    w = c * NS + s
    @pl.when(s < N // RB)
    def _():
        pltpu.sync_copy(zeros_hbm, acc_sh.at[pl.ds(s * RB, RB)])
    pltpu.sync_copy(ones_hbm, ones_v)
    plsc.subcore_barrier()

    def chunk(j, carry):
        off = w * EPW + j * CH
        pltpu.sync_copy(dst_hbm.at[pl.ds(off, CH)], idx_v)
        pltpu.sync_copy(ones_v, acc_sh.at[idx_v], add=True)
        return carry

    lax.fori_loop(0, NCH, chunk, 0)
    plsc.subcore_barrier()

    @pl.when(s < N // RB)
    def _():
        pltpu.sync_copy(acc_sh.at[pl.ds(s * RB, RB)],
                        out_hbm.at[c, pl.ds(s * RB, RB)])


_deg_call = pl.kernel(
    _deg_body,
    out_type=jax.ShapeDtypeStruct((NC, N, D), jnp.float32),
    mesh=_mesh,
    scratch_types=[
        pltpu.VMEM((CH, D), jnp.float32),
        pltpu.VMEM((CH,), jnp.int32),
        pltpu.VMEM_SHARED((N, D), jnp.float32),
    ],
)


NB = 4
NG = NCH // NB


def _spmm_body(xws_hbm, src_hbm, dst_hbm, zeros_hbm, out_hbm, *refs):
    idx_s = refs[0:NB]
    idx_d = refs[NB:2 * NB]
    rows = refs[2 * NB:3 * NB]
    isem = refs[3 * NB]
    gsem = refs[3 * NB + 1:3 * NB + 1 + NB]
    ssem = refs[3 * NB + 1 + NB:3 * NB + 1 + 2 * NB]
    acc_sh = refs[3 * NB + 1 + 2 * NB]
    c = lax.axis_index("c")
    s = lax.axis_index("s")
    w = c * NS + s

    @pl.when(s < N // RB)
    def _():
        pltpu.sync_copy(zeros_hbm, acc_sh.at[pl.ds(s * RB, RB)])

    plsc.subcore_barrier()

    def group(g, carry):
        goff = w * EPW + g * (NB * CH)
        di = []
        for b in range(NB):
            off = goff + b * CH
            d1 = pltpu.async_copy(src_hbm.at[pl.ds(off, CH)], idx_s[b], isem)
            d2 = pltpu.async_copy(dst_hbm.at[pl.ds(off, CH)], idx_d[b], isem)
            di.append((d1, d2))
        gd = []
        for b in range(NB):
            di[b][0].wait()
            di[b][1].wait()
            gd.append(pltpu.async_copy(xws_hbm.at[idx_s[b]], rows[b], gsem[b]))
        sd = []
        for b in range(NB):
            gd[b].wait()
            sd.append(pltpu.async_copy(rows[b], acc_sh.at[idx_d[b]],
                                       ssem[b], add=True))
        for d in sd:
            d.wait()
        return carry

    lax.fori_loop(0, NG, group, 0)
    for r in range(NCH % NB):
        off = w * EPW + (NG * NB + r) * CH
        pltpu.sync_copy(src_hbm.at[pl.ds(off, CH)], idx_s[0])
        pltpu.sync_copy(dst_hbm.at[pl.ds(off, CH)], idx_d[0])
        pltpu.async_copy(xws_hbm.at[idx_s[0]], rows[0], gsem[0]).wait()
        pltpu.sync_copy(rows[0], acc_sh.at[idx_d[0]], add=True)
    plsc.subcore_barrier()

    @pl.when(s < N // RB)
    def _():
        pltpu.sync_copy(acc_sh.at[pl.ds(s * RB, RB)],
                        out_hbm.at[c, pl.ds(s * RB, RB)])


_spmm_call = pl.kernel(
    _spmm_body,
    out_type=jax.ShapeDtypeStruct((NC, N, D), jnp.float32),
    mesh=_mesh,
    scratch_types=(
        [pltpu.VMEM((CH,), jnp.int32) for _ in range(2 * NB)]
        + [pltpu.VMEM((CH, D), jnp.float32) for _ in range(NB)]
        + [pltpu.SemaphoreType.DMA for _ in range(2 * NB + 1)]
        + [pltpu.VMEM_SHARED((N, D), jnp.float32)]
    ),
)


_R = 1000


def _dis(d0_ref, d1_ref):
    deg = d0_ref[:, 0:1] + d1_ref[:, 0:1] + 1.0
    return lax.rsqrt(deg)


def _pre_body(x_ref, w_ref, d0_ref, d1_ref, o_ref):
    dis = _dis(d0_ref, d1_ref)
    xw = jnp.dot(x_ref[...], w_ref[...], preferred_element_type=jnp.float32)
    o_ref[...] = dis * xw


def _mid_body(a0_ref, a1_ref, xws_ref, d0_ref, d1_ref, b_ref, w_ref, o_ref):
    dis = _dis(d0_ref, d1_ref)
    pre = dis * (a0_ref[...] + a1_ref[...] + xws_ref[...]) + b_ref[...]
    h = jnp.maximum(pre, 0.0)
    o_ref[...] = dis * jnp.dot(h, w_ref[...],
                               preferred_element_type=jnp.float32)


def _post_body(a0_ref, a1_ref, xws_ref, d0_ref, d1_ref, b_ref, o_ref):
    dis = _dis(d0_ref, d1_ref)
    o_ref[...] = dis * (a0_ref[...] + a1_ref[...] + xws_ref[...]) + b_ref[...]


_row_spec = pl.BlockSpec((_R, D), lambda i: (i, 0))
_deg_spec = pl.BlockSpec((_R, D), lambda i: (i, 0))
_w_spec = pl.BlockSpec((D, D), lambda i: (0, 0))
_b_spec = pl.BlockSpec((1, D), lambda i: (0, 0))
_grid = (N // _R,)

_pre_call = pl.pallas_call(
    _pre_body,
    grid=_grid,
    in_specs=[_row_spec, _w_spec, _deg_spec, _deg_spec],
    out_specs=_row_spec,
    out_shape=jax.ShapeDtypeStruct((N, D), jnp.float32),
)

_mid_call = pl.pallas_call(
    _mid_body,
    grid=_grid,
    in_specs=[_row_spec, _row_spec, _row_spec, _deg_spec, _deg_spec,
              _b_spec, _w_spec],
    out_specs=_row_spec,
    out_shape=jax.ShapeDtypeStruct((N, D), jnp.float32),
)

_post_call = pl.pallas_call(
    _post_body,
    grid=_grid,
    in_specs=[_row_spec, _row_spec, _row_spec, _deg_spec, _deg_spec, _b_spec],
    out_specs=_row_spec,
    out_shape=jax.ShapeDtypeStruct((N, D), jnp.float32),
)


def kernel(x, edge_index, W1, b1, W2, b2):
    ei = edge_index.astype(jnp.int32)
    src, dst = ei[0], ei[1]
    onesD = jnp.ones((CH, D), jnp.float32)
    zerosD = jnp.zeros((RB, D), jnp.float32)
    b1r = b1.reshape(1, D)
    b2r = b2.reshape(1, D)

    degp = _deg_call(dst, onesD, zerosD)
    d0, d1 = degp[0], degp[1]

    xw1s = _pre_call(x, W1, d0, d1)
    acc1 = _spmm_call(xw1s, src, dst, zerosD)
    xw2s = _mid_call(acc1[0], acc1[1], xw1s, d0, d1, b1r, W2)
    acc2 = _spmm_call(xw2s, src, dst, zerosD)
    return _post_call(acc2[0], acc2[1], xw2s, d0, d1, b2r)
